# baseline (speedup 1.0000x reference)
import functools

import jax
import jax.numpy as jnp
from jax import lax
from jax.experimental import pallas as pl
from jax.experimental.pallas import tpu as pltpu
from jax.experimental.pallas import tpu_sc as plsc

E = 8
K = 2
D = 768
F = 3072
T = 2048
NP = T * K

M = 256
NB = NP // M + E
PN = NB * M

CS = 512
NCH = NP // CS

FCHUNK = 1024
NF = F // FCHUNK

NWORK = 32
CH = T // NWORK


def _router_body(x_ref, wg_ref, bg_ref,
                 logits_ref, pos_ref, meta_ref, wrows_ref):
    x = x_ref[...]
    logits = lax.dot_general(
        x, wg_ref[...], (((1,), (0,)), ((), ())),
        preferred_element_type=jnp.float32) + bg_ref[...]
    logits_ref[...] = logits

    iota = lax.broadcasted_iota(jnp.int32, (T, E), 1)
    m0 = jnp.max(logits, axis=1, keepdims=True)
    e0 = jnp.min(jnp.where(logits == m0, iota, E), axis=1, keepdims=True)
    masked = jnp.where(iota == e0, -jnp.inf, logits)
    m1 = jnp.max(masked, axis=1, keepdims=True)
    e1 = jnp.min(jnp.where(masked == m1, iota, E), axis=1, keepdims=True)
    t = jnp.exp(m1 - m0)
    w0 = 1.0 / (1.0 + t)
    w1 = t / (1.0 + t)

    wrows_ref[0:T, :] = jnp.broadcast_to(w0, (T, 16))
    wrows_ref[T:NP, :] = jnp.broadcast_to(w1, (T, 16))

    r_i = lax.broadcasted_iota(jnp.int32, (CS, CS), 0)
    s_i = lax.broadcasted_iota(jnp.int32, (CS, CS), 1)
    tril = (s_i < r_i).astype(jnp.bfloat16)
    iota_c = lax.broadcasted_iota(jnp.int32, (CS, E), 1)

    counts = jnp.zeros((1, E), jnp.float32)
    rank_cols = [[], []]
    for c in range(NCH):
        kk = c // (T // CS)
        tt = (c % (T // CS)) * CS
        ec = (e0 if kk == 0 else e1)[tt:tt + CS]
        onehot = iota_c == ec
        oh_b = onehot.astype(jnp.bfloat16)
        prefix = lax.dot_general(
            tril, oh_b, (((1,), (0,)), ((), ())),
            preferred_element_type=jnp.float32) + counts
        rank_cols[kk].append(
            jnp.sum(jnp.where(onehot, prefix, 0.0), axis=1, keepdims=True))
        counts = counts + jnp.sum(onehot.astype(jnp.float32), axis=0,
                                  keepdims=True)
    rank0 = jnp.concatenate(rank_cols[0], axis=0)
    rank1 = jnp.concatenate(rank_cols[1], axis=0)

    counts_i = counts.astype(jnp.int32)
    nb = (counts_i + (M - 1)) // M
    e_ri = lax.broadcasted_iota(jnp.int32, (E, E), 0)
    e_si = lax.broadcasted_iota(jnp.int32, (E, E), 1)
    tril8 = (e_ri < e_si).astype(jnp.float32)
    startblk = lax.dot_general(
        nb.astype(jnp.float32), tril8, (((1,), (0,)), ((), ())),
        preferred_element_type=jnp.float32)
    offset = startblk * M
    totalblk = jnp.sum(nb)

    off0 = jnp.sum(jnp.where(iota == e0, offset, 0.0), axis=1, keepdims=True)
    off1 = jnp.sum(jnp.where(iota == e1, offset, 0.0), axis=1, keepdims=True)
    pos0 = off0 + rank0
    pos1 = off1 + rank1
    pos_ref[...] = jnp.concatenate([pos0, pos1], axis=1).astype(jnp.int32)

    b_i = lax.broadcasted_iota(jnp.int32, (NB, E), 0)
    be = jnp.sum((b_i >= startblk.astype(jnp.int32)).astype(jnp.int32),
                 axis=1, keepdims=True) - 1
    be = jnp.clip(be, 0, E - 1)
    valid = (lax.broadcasted_iota(jnp.int32, (NB, 1), 0)
             < totalblk).astype(jnp.int32)
    meta_ref[...] = jnp.concatenate([be, valid], axis=1)


def _dispatch_body(x_hbm, posf_hbm, xs_hbm, xbuf, idx, s0, s1):
    wid = lax.axis_index("s") * 2 + lax.axis_index("c")
    base = wid * CH
    pltpu.sync_copy(posf_hbm.at[0, pl.ds(base, CH)], idx.at[0])
    pltpu.sync_copy(posf_hbm.at[1, pl.ds(base, CH)], idx.at[1])
    pltpu.sync_copy(x_hbm.at[pl.ds(base, CH)], xbuf)
    c0 = pltpu.async_copy(xbuf, xs_hbm.at[idx.at[0]], s0)
    c1 = pltpu.async_copy(xbuf, xs_hbm.at[idx.at[1]], s1)
    c0.wait()
    c1.wait()


def _group_mm_body(meta_ref, xs_ref, w1_ref, b1_ref, w2_ref, b2_ref,
                   ys_ref, w1bf_ref, w2bf_ref):
    b = pl.program_id(0)
    valid = meta_ref[b, 1]
    e_cur = meta_ref[b, 0]
    e_prev = meta_ref[jnp.maximum(b - 1, 0), 0]
    fresh = jnp.logical_or(b == 0, e_cur != e_prev)

    @pl.when(jnp.logical_and(valid == 1, fresh))
    def _():
        w1bf_ref[...] = w1_ref[0].astype(jnp.bfloat16)
        w2bf_ref[...] = w2_ref[0].astype(jnp.bfloat16)

    @pl.when(valid == 1)
    def _():
        xs = xs_ref[...].astype(jnp.bfloat16)
        y = b2_ref[0] + jnp.zeros((M, D), jnp.float32)
        for f in range(NF):
            h = lax.dot_general(
                xs, w1bf_ref[:, f * FCHUNK:(f + 1) * FCHUNK],
                (((1,), (0,)), ((), ())),
                preferred_element_type=jnp.float32)
            h = jnp.maximum(h + b1_ref[0, :, f * FCHUNK:(f + 1) * FCHUNK],
                            0.0).astype(jnp.bfloat16)
            y = y + lax.dot_general(
                h, w2bf_ref[f * FCHUNK:(f + 1) * FCHUNK, :],
                (((1,), (0,)), ((), ())),
                preferred_element_type=jnp.float32)
        ys_ref[...] = y


def _combine_body(posf_hbm, ys_hbm, wrows_hbm, out_hbm,
                  idx0, idx1, ya, yb, wbuf0, wbuf1, s0, s1):
    wid = lax.axis_index("s") * 2 + lax.axis_index("c")
    base = wid * CH
    pltpu.sync_copy(posf_hbm.at[0, pl.ds(base, CH)], idx0)
    pltpu.sync_copy(posf_hbm.at[1, pl.ds(base, CH)], idx1)
    c0 = pltpu.async_copy(ys_hbm.at[idx0], ya, s0)
    c1 = pltpu.async_copy(ys_hbm.at[idx1], yb, s1)
    pltpu.sync_copy(wrows_hbm.at[pl.ds(base, CH)], wbuf0)
    pltpu.sync_copy(wrows_hbm.at[pl.ds(T + base, CH)], wbuf1)
    c0.wait()
    c1.wait()

    @pl.loop(0, CH)
    def _(r):
        wa = wbuf0[r, pl.ds(0, 16)]
        wb = wbuf1[r, pl.ds(0, 16)]

        @pl.loop(0, D, step=16)
        def _(c):
            ya[r, pl.ds(c, 16)] = (wa * ya[r, pl.ds(c, 16)]
                                   + wb * yb[r, pl.ds(c, 16)])

    pltpu.sync_copy(ya, out_hbm.at[pl.ds(base, CH)])


@jax.jit
def kernel(x, Wg, bg, W1, b1, W2, b2):
    x_flat = x.reshape(T, D)
    logits, pos, meta, wrows = pl.pallas_call(
        _router_body,
        out_shape=(
            jax.ShapeDtypeStruct((T, E), jnp.float32),
            jax.ShapeDtypeStruct((T, K), jnp.int32),
            jax.ShapeDtypeStruct((NB, 2), jnp.int32),
            jax.ShapeDtypeStruct((NP, 16), jnp.float32),
        ),
    )(x_flat, Wg, bg.reshape(1, E))

    posf = pos.T.reshape(K, T)

    mesh = plsc.VectorSubcoreMesh(core_axis_name="c", subcore_axis_name="s")
    xs = pl.kernel(
        _dispatch_body,
        mesh=mesh,
        out_type=jax.ShapeDtypeStruct((PN, D), jnp.float32),
        scratch_types=[
            pltpu.VMEM((CH, D), jnp.float32),
            pltpu.VMEM((K, CH), jnp.int32),
            pltpu.SemaphoreType.DMA,
            pltpu.SemaphoreType.DMA,
        ],
    )(x_flat, posf)

    ys = pl.pallas_call(
        _group_mm_body,
        grid_spec=pltpu.PrefetchScalarGridSpec(
            num_scalar_prefetch=1,
            grid=(NB,),
            in_specs=[
                pl.BlockSpec((M, D), lambda b, m: (b, 0)),
                pl.BlockSpec((1, D, F), lambda b, m: (m[b, 0], 0, 0)),
                pl.BlockSpec((1, 1, F), lambda b, m: (m[b, 0], 0, 0)),
                pl.BlockSpec((1, F, D), lambda b, m: (m[b, 0], 0, 0)),
                pl.BlockSpec((1, 1, D), lambda b, m: (m[b, 0], 0, 0)),
            ],
            out_specs=pl.BlockSpec((M, D), lambda b, m: (b, 0)),
            scratch_shapes=[
                pltpu.VMEM((D, F), jnp.bfloat16),
                pltpu.VMEM((F, D), jnp.bfloat16),
            ],
        ),
        out_shape=jax.ShapeDtypeStruct((PN, D), jnp.float32),
        compiler_params=pltpu.CompilerParams(
            dimension_semantics=("parallel",)),
    )(meta, xs, W1, b1.reshape(E, 1, F), W2, b2.reshape(E, 1, D))

    out = pl.kernel(
        _combine_body,
        mesh=mesh,
        out_type=jax.ShapeDtypeStruct((T, D), jnp.float32),
        scratch_types=[
            pltpu.VMEM((CH,), jnp.int32),
            pltpu.VMEM((CH,), jnp.int32),
            pltpu.VMEM((CH, D), jnp.float32),
            pltpu.VMEM((CH, D), jnp.float32),
            pltpu.VMEM((CH, 16), jnp.float32),
            pltpu.VMEM((CH, 16), jnp.float32),
            pltpu.SemaphoreType.DMA,
            pltpu.SemaphoreType.DMA,
        ],
    )(posf, ys, wrows)

    return out.reshape(1, T, D), logits


# speedup vs baseline: 1.1276x; 1.1276x over previous
import functools

import jax
import jax.numpy as jnp
from jax import lax
from jax.experimental import pallas as pl
from jax.experimental.pallas import tpu as pltpu
from jax.experimental.pallas import tpu_sc as plsc

E = 8
K = 2
D = 768
F = 3072
T = 2048
NP = T * K

M = 256
NB = NP // M + E
PN = NB * M

CS = 512
NCH = NP // CS

FCHUNK = 1024
NF = F // FCHUNK

NWORK = 32
CH = T // NWORK


def _router_body(x_ref, wg_ref, bg_ref,
                 logits_ref, pos_ref, meta_ref, wrows_ref):
    x = x_ref[...]
    logits = lax.dot_general(
        x, wg_ref[...], (((1,), (0,)), ((), ())),
        preferred_element_type=jnp.float32) + bg_ref[...]
    logits_ref[...] = logits

    iota = lax.broadcasted_iota(jnp.int32, (T, E), 1)
    m0 = jnp.max(logits, axis=1, keepdims=True)
    e0 = jnp.min(jnp.where(logits == m0, iota, E), axis=1, keepdims=True)
    masked = jnp.where(iota == e0, -jnp.inf, logits)
    m1 = jnp.max(masked, axis=1, keepdims=True)
    e1 = jnp.min(jnp.where(masked == m1, iota, E), axis=1, keepdims=True)
    t = jnp.exp(m1 - m0)
    w0 = 1.0 / (1.0 + t)
    w1 = t / (1.0 + t)

    wrows_ref[0:T, :] = jnp.broadcast_to(w0, (T, 16))
    wrows_ref[T:NP, :] = jnp.broadcast_to(w1, (T, 16))

    r_i = lax.broadcasted_iota(jnp.int32, (CS, CS), 0)
    s_i = lax.broadcasted_iota(jnp.int32, (CS, CS), 1)
    tril = (s_i < r_i).astype(jnp.bfloat16)
    iota_c = lax.broadcasted_iota(jnp.int32, (CS, E), 1)

    counts = jnp.zeros((1, E), jnp.float32)
    rank_cols = [[], []]
    for c in range(NCH):
        kk = c // (T // CS)
        tt = (c % (T // CS)) * CS
        ec = (e0 if kk == 0 else e1)[tt:tt + CS]
        onehot = iota_c == ec
        oh_b = onehot.astype(jnp.bfloat16)
        prefix = lax.dot_general(
            tril, oh_b, (((1,), (0,)), ((), ())),
            preferred_element_type=jnp.float32) + counts
        rank_cols[kk].append(
            jnp.sum(jnp.where(onehot, prefix, 0.0), axis=1, keepdims=True))
        counts = counts + jnp.sum(onehot.astype(jnp.float32), axis=0,
                                  keepdims=True)
    rank0 = jnp.concatenate(rank_cols[0], axis=0)
    rank1 = jnp.concatenate(rank_cols[1], axis=0)

    counts_i = counts.astype(jnp.int32)
    nb = (counts_i + (M - 1)) // M
    e_ri = lax.broadcasted_iota(jnp.int32, (E, E), 0)
    e_si = lax.broadcasted_iota(jnp.int32, (E, E), 1)
    tril8 = (e_ri < e_si).astype(jnp.float32)
    startblk = lax.dot_general(
        nb.astype(jnp.float32), tril8, (((1,), (0,)), ((), ())),
        preferred_element_type=jnp.float32)
    offset = startblk * M
    totalblk = jnp.sum(nb)

    off0 = jnp.sum(jnp.where(iota == e0, offset, 0.0), axis=1, keepdims=True)
    off1 = jnp.sum(jnp.where(iota == e1, offset, 0.0), axis=1, keepdims=True)
    pos0 = off0 + rank0
    pos1 = off1 + rank1
    pos_ref[...] = jnp.concatenate([pos0, pos1], axis=1).astype(jnp.int32)

    b_i = lax.broadcasted_iota(jnp.int32, (NB, E), 0)
    be = jnp.sum((b_i >= startblk.astype(jnp.int32)).astype(jnp.int32),
                 axis=1, keepdims=True) - 1
    be = jnp.clip(be, 0, E - 1)
    valid = (lax.broadcasted_iota(jnp.int32, (NB, 1), 0)
             < totalblk).astype(jnp.int32)
    meta_ref[...] = jnp.concatenate([be, valid], axis=1)


def _dispatch_body(x_hbm, posf_hbm, xs_hbm, xbuf, idx, s0, s1):
    wid = lax.axis_index("s") * 2 + lax.axis_index("c")
    base = wid * CH
    pltpu.sync_copy(posf_hbm.at[0, pl.ds(base, CH)], idx.at[0])
    pltpu.sync_copy(posf_hbm.at[1, pl.ds(base, CH)], idx.at[1])
    pltpu.sync_copy(x_hbm.at[pl.ds(base, CH)], xbuf)
    c0 = pltpu.async_copy(xbuf, xs_hbm.at[idx.at[0]], s0)
    c1 = pltpu.async_copy(xbuf, xs_hbm.at[idx.at[1]], s1)
    c0.wait()
    c1.wait()


def _group_mm_body(meta_ref, xs_ref, w1_ref, b1_ref, w2_ref, b2_ref,
                   ys_ref):
    b = pl.program_id(0)
    valid = meta_ref[b, 1]

    @pl.when(valid == 1)
    def _():
        xs = xs_ref[...].astype(jnp.bfloat16)
        y = b2_ref[0] + jnp.zeros((M, D), jnp.float32)
        for f in range(NF):
            w1 = w1_ref[0, :, f * FCHUNK:(f + 1) * FCHUNK].astype(jnp.bfloat16)
            h = lax.dot_general(
                xs, w1, (((1,), (0,)), ((), ())),
                preferred_element_type=jnp.float32)
            h = jnp.maximum(h + b1_ref[0, :, f * FCHUNK:(f + 1) * FCHUNK],
                            0.0).astype(jnp.bfloat16)
            w2 = w2_ref[0, f * FCHUNK:(f + 1) * FCHUNK, :].astype(jnp.bfloat16)
            y = y + lax.dot_general(
                h, w2, (((1,), (0,)), ((), ())),
                preferred_element_type=jnp.float32)
        ys_ref[...] = y


def _combine_body(posf_hbm, ys_hbm, wrows_hbm, out_hbm,
                  idx0, idx1, ya, yb, wbuf0, wbuf1, s0, s1):
    wid = lax.axis_index("s") * 2 + lax.axis_index("c")
    base = wid * CH
    pltpu.sync_copy(posf_hbm.at[0, pl.ds(base, CH)], idx0)
    pltpu.sync_copy(posf_hbm.at[1, pl.ds(base, CH)], idx1)
    c0 = pltpu.async_copy(ys_hbm.at[idx0], ya, s0)
    c1 = pltpu.async_copy(ys_hbm.at[idx1], yb, s1)
    pltpu.sync_copy(wrows_hbm.at[pl.ds(base, CH)], wbuf0)
    pltpu.sync_copy(wrows_hbm.at[pl.ds(T + base, CH)], wbuf1)
    c0.wait()
    c1.wait()

    @pl.loop(0, CH)
    def _(r):
        wa = wbuf0[r, pl.ds(0, 16)]
        wb = wbuf1[r, pl.ds(0, 16)]

        @plsc.parallel_loop(0, D, step=16, unroll=4)
        def _(c):
            ya[r, pl.ds(c, 16)] = (wa * ya[r, pl.ds(c, 16)]
                                   + wb * yb[r, pl.ds(c, 16)])

    pltpu.sync_copy(ya, out_hbm.at[pl.ds(base, CH)])


@jax.jit
def kernel(x, Wg, bg, W1, b1, W2, b2):
    x_flat = x.reshape(T, D)
    logits, pos, meta, wrows = pl.pallas_call(
        _router_body,
        out_shape=(
            jax.ShapeDtypeStruct((T, E), jnp.float32),
            jax.ShapeDtypeStruct((T, K), jnp.int32),
            jax.ShapeDtypeStruct((NB, 2), jnp.int32),
            jax.ShapeDtypeStruct((NP, 16), jnp.float32),
        ),
    )(x_flat, Wg, bg.reshape(1, E))

    posf = pos.T.reshape(K, T)

    mesh = plsc.VectorSubcoreMesh(core_axis_name="c", subcore_axis_name="s")
    xs = pl.kernel(
        _dispatch_body,
        mesh=mesh,
        out_type=jax.ShapeDtypeStruct((PN, D), jnp.float32),
        scratch_types=[
            pltpu.VMEM((CH, D), jnp.float32),
            pltpu.VMEM((K, CH), jnp.int32),
            pltpu.SemaphoreType.DMA,
            pltpu.SemaphoreType.DMA,
        ],
    )(x_flat, posf)

    ys = pl.pallas_call(
        _group_mm_body,
        grid_spec=pltpu.PrefetchScalarGridSpec(
            num_scalar_prefetch=1,
            grid=(NB,),
            in_specs=[
                pl.BlockSpec((M, D), lambda b, m: (b, 0)),
                pl.BlockSpec((1, D, F), lambda b, m: (m[b, 0], 0, 0)),
                pl.BlockSpec((1, 1, F), lambda b, m: (m[b, 0], 0, 0)),
                pl.BlockSpec((1, F, D), lambda b, m: (m[b, 0], 0, 0)),
                pl.BlockSpec((1, 1, D), lambda b, m: (m[b, 0], 0, 0)),
            ],
            out_specs=pl.BlockSpec((M, D), lambda b, m: (b, 0)),
        ),
        out_shape=jax.ShapeDtypeStruct((PN, D), jnp.float32),
        compiler_params=pltpu.CompilerParams(
            dimension_semantics=("parallel",)),
    )(meta, xs, W1, b1.reshape(E, 1, F), W2, b2.reshape(E, 1, D))

    out = pl.kernel(
        _combine_body,
        mesh=mesh,
        out_type=jax.ShapeDtypeStruct((T, D), jnp.float32),
        scratch_types=[
            pltpu.VMEM((CH,), jnp.int32),
            pltpu.VMEM((CH,), jnp.int32),
            pltpu.VMEM((CH, D), jnp.float32),
            pltpu.VMEM((CH, D), jnp.float32),
            pltpu.VMEM((CH, 16), jnp.float32),
            pltpu.VMEM((CH, 16), jnp.float32),
            pltpu.SemaphoreType.DMA,
            pltpu.SemaphoreType.DMA,
        ],
    )(posf, ys, wrows)

    return out.reshape(1, T, D), logits


# speedup vs baseline: 1.2277x; 1.0888x over previous
import functools

import jax
import jax.numpy as jnp
from jax import lax
from jax.experimental import pallas as pl
from jax.experimental.pallas import tpu as pltpu
from jax.experimental.pallas import tpu_sc as plsc

E = 8
K = 2
D = 768
F = 3072
T = 2048
NP = T * K

M = 512
NB = NP // M + E
PN = NB * M

CS = 512
NCH = NP // CS

FCHUNK = 1024
NF = F // FCHUNK

NWORK = 32
CH = T // NWORK


def _router_body(x_ref, wg_ref, bg_ref,
                 logits_ref, pos_ref, meta_ref, wrows_ref):
    x = x_ref[...]
    logits = lax.dot_general(
        x, wg_ref[...], (((1,), (0,)), ((), ())),
        preferred_element_type=jnp.float32) + bg_ref[...]
    logits_ref[...] = logits

    iota = lax.broadcasted_iota(jnp.int32, (T, E), 1)
    m0 = jnp.max(logits, axis=1, keepdims=True)
    e0 = jnp.min(jnp.where(logits == m0, iota, E), axis=1, keepdims=True)
    masked = jnp.where(iota == e0, -jnp.inf, logits)
    m1 = jnp.max(masked, axis=1, keepdims=True)
    e1 = jnp.min(jnp.where(masked == m1, iota, E), axis=1, keepdims=True)
    t = jnp.exp(m1 - m0)
    w0 = 1.0 / (1.0 + t)
    w1 = t / (1.0 + t)

    wrows_ref[0:T, :] = jnp.broadcast_to(w0, (T, 16))
    wrows_ref[T:NP, :] = jnp.broadcast_to(w1, (T, 16))

    r_i = lax.broadcasted_iota(jnp.int32, (CS, CS), 0)
    s_i = lax.broadcasted_iota(jnp.int32, (CS, CS), 1)
    tril = (s_i < r_i).astype(jnp.bfloat16)
    iota_c = lax.broadcasted_iota(jnp.int32, (CS, E), 1)

    counts = jnp.zeros((1, E), jnp.float32)
    rank_cols = [[], []]
    for c in range(NCH):
        kk = c // (T // CS)
        tt = (c % (T // CS)) * CS
        ec = (e0 if kk == 0 else e1)[tt:tt + CS]
        onehot = iota_c == ec
        oh_b = onehot.astype(jnp.bfloat16)
        prefix = lax.dot_general(
            tril, oh_b, (((1,), (0,)), ((), ())),
            preferred_element_type=jnp.float32) + counts
        rank_cols[kk].append(
            jnp.sum(jnp.where(onehot, prefix, 0.0), axis=1, keepdims=True))
        counts = counts + jnp.sum(onehot.astype(jnp.float32), axis=0,
                                  keepdims=True)
    rank0 = jnp.concatenate(rank_cols[0], axis=0)
    rank1 = jnp.concatenate(rank_cols[1], axis=0)

    counts_i = counts.astype(jnp.int32)
    nb = (counts_i + (M - 1)) // M
    e_ri = lax.broadcasted_iota(jnp.int32, (E, E), 0)
    e_si = lax.broadcasted_iota(jnp.int32, (E, E), 1)
    tril8 = (e_ri < e_si).astype(jnp.float32)
    startblk = lax.dot_general(
        nb.astype(jnp.float32), tril8, (((1,), (0,)), ((), ())),
        preferred_element_type=jnp.float32)
    offset = startblk * M
    totalblk = jnp.sum(nb)

    off0 = jnp.sum(jnp.where(iota == e0, offset, 0.0), axis=1, keepdims=True)
    off1 = jnp.sum(jnp.where(iota == e1, offset, 0.0), axis=1, keepdims=True)
    pos0 = off0 + rank0
    pos1 = off1 + rank1
    pos_ref[...] = jnp.concatenate([pos0, pos1], axis=1).astype(jnp.int32)

    b_i = lax.broadcasted_iota(jnp.int32, (NB, E), 0)
    be = jnp.sum((b_i >= startblk.astype(jnp.int32)).astype(jnp.int32),
                 axis=1, keepdims=True) - 1
    be = jnp.clip(be, 0, E - 1)
    valid = (lax.broadcasted_iota(jnp.int32, (NB, 1), 0)
             < totalblk).astype(jnp.int32)
    meta_ref[...] = jnp.concatenate([be, valid], axis=1)


def _dispatch_body(x_hbm, posf_hbm, xs_hbm, xbuf, idx, s0, s1):
    wid = lax.axis_index("s") * 2 + lax.axis_index("c")
    base = wid * CH
    pltpu.sync_copy(posf_hbm.at[0, pl.ds(base, CH)], idx.at[0])
    pltpu.sync_copy(posf_hbm.at[1, pl.ds(base, CH)], idx.at[1])
    pltpu.sync_copy(x_hbm.at[pl.ds(base, CH)], xbuf)
    c0 = pltpu.async_copy(xbuf, xs_hbm.at[idx.at[0]], s0)
    c1 = pltpu.async_copy(xbuf, xs_hbm.at[idx.at[1]], s1)
    c0.wait()
    c1.wait()


def _group_mm_body(meta_ref, xs_ref, w1_ref, b1_ref, w2_ref, b2_ref,
                   ys_ref):
    b = pl.program_id(0)
    valid = meta_ref[b, 1]

    @pl.when(valid == 1)
    def _():
        xs = xs_ref[...].astype(jnp.bfloat16)
        y = b2_ref[0] + jnp.zeros((M, D), jnp.float32)
        for f in range(NF):
            w1 = w1_ref[0, :, f * FCHUNK:(f + 1) * FCHUNK].astype(jnp.bfloat16)
            h = lax.dot_general(
                xs, w1, (((1,), (0,)), ((), ())),
                preferred_element_type=jnp.float32)
            h = jnp.maximum(h + b1_ref[0, :, f * FCHUNK:(f + 1) * FCHUNK],
                            0.0).astype(jnp.bfloat16)
            w2 = w2_ref[0, f * FCHUNK:(f + 1) * FCHUNK, :].astype(jnp.bfloat16)
            y = y + lax.dot_general(
                h, w2, (((1,), (0,)), ((), ())),
                preferred_element_type=jnp.float32)
        ys_ref[...] = y


def _combine_body(posf_hbm, ys_hbm, wrows_hbm, out_hbm,
                  idx0, idx1, ya, yb, wbuf0, wbuf1, s0, s1):
    wid = lax.axis_index("s") * 2 + lax.axis_index("c")
    base = wid * CH
    pltpu.sync_copy(posf_hbm.at[0, pl.ds(base, CH)], idx0)
    pltpu.sync_copy(posf_hbm.at[1, pl.ds(base, CH)], idx1)
    c0 = pltpu.async_copy(ys_hbm.at[idx0], ya, s0)
    c1 = pltpu.async_copy(ys_hbm.at[idx1], yb, s1)
    pltpu.sync_copy(wrows_hbm.at[pl.ds(base, CH)], wbuf0)
    pltpu.sync_copy(wrows_hbm.at[pl.ds(T + base, CH)], wbuf1)
    c0.wait()
    c1.wait()

    @pl.loop(0, CH)
    def _(r):
        wa = wbuf0[r, pl.ds(0, 16)]
        wb = wbuf1[r, pl.ds(0, 16)]

        @plsc.parallel_loop(0, D, step=16, unroll=4)
        def _(c):
            ya[r, pl.ds(c, 16)] = (wa * ya[r, pl.ds(c, 16)]
                                   + wb * yb[r, pl.ds(c, 16)])

    pltpu.sync_copy(ya, out_hbm.at[pl.ds(base, CH)])


@jax.jit
def kernel(x, Wg, bg, W1, b1, W2, b2):
    x_flat = x.reshape(T, D)
    logits, pos, meta, wrows = pl.pallas_call(
        _router_body,
        out_shape=(
            jax.ShapeDtypeStruct((T, E), jnp.float32),
            jax.ShapeDtypeStruct((T, K), jnp.int32),
            jax.ShapeDtypeStruct((NB, 2), jnp.int32),
            jax.ShapeDtypeStruct((NP, 16), jnp.float32),
        ),
    )(x_flat, Wg, bg.reshape(1, E))

    posf = pos.T.reshape(K, T)

    mesh = plsc.VectorSubcoreMesh(core_axis_name="c", subcore_axis_name="s")
    xs = pl.kernel(
        _dispatch_body,
        mesh=mesh,
        out_type=jax.ShapeDtypeStruct((PN, D), jnp.float32),
        scratch_types=[
            pltpu.VMEM((CH, D), jnp.float32),
            pltpu.VMEM((K, CH), jnp.int32),
            pltpu.SemaphoreType.DMA,
            pltpu.SemaphoreType.DMA,
        ],
    )(x_flat, posf)

    ys = pl.pallas_call(
        _group_mm_body,
        grid_spec=pltpu.PrefetchScalarGridSpec(
            num_scalar_prefetch=1,
            grid=(NB,),
            in_specs=[
                pl.BlockSpec((M, D), lambda b, m: (b, 0)),
                pl.BlockSpec((1, D, F), lambda b, m: (m[b, 0], 0, 0)),
                pl.BlockSpec((1, 1, F), lambda b, m: (m[b, 0], 0, 0)),
                pl.BlockSpec((1, F, D), lambda b, m: (m[b, 0], 0, 0)),
                pl.BlockSpec((1, 1, D), lambda b, m: (m[b, 0], 0, 0)),
            ],
            out_specs=pl.BlockSpec((M, D), lambda b, m: (b, 0)),
        ),
        out_shape=jax.ShapeDtypeStruct((PN, D), jnp.float32),
        compiler_params=pltpu.CompilerParams(
            dimension_semantics=("parallel",)),
    )(meta, xs, W1, b1.reshape(E, 1, F), W2, b2.reshape(E, 1, D))

    out = pl.kernel(
        _combine_body,
        mesh=mesh,
        out_type=jax.ShapeDtypeStruct((T, D), jnp.float32),
        scratch_types=[
            pltpu.VMEM((CH,), jnp.int32),
            pltpu.VMEM((CH,), jnp.int32),
            pltpu.VMEM((CH, D), jnp.float32),
            pltpu.VMEM((CH, D), jnp.float32),
            pltpu.VMEM((CH, 16), jnp.float32),
            pltpu.VMEM((CH, 16), jnp.float32),
            pltpu.SemaphoreType.DMA,
            pltpu.SemaphoreType.DMA,
        ],
    )(posf, ys, wrows)

    return out.reshape(1, T, D), logits


# speedup vs baseline: 1.3061x; 1.0639x over previous
import functools

import jax
import jax.numpy as jnp
from jax import lax
from jax.experimental import pallas as pl
from jax.experimental.pallas import tpu as pltpu
from jax.experimental.pallas import tpu_sc as plsc

E = 8
K = 2
D = 768
F = 3072
T = 2048
NP = T * K

M = 512
NB = NP // M + E
PN = NB * M

CS = 512
NCH = NP // CS

FCHUNK = 1024
NF = F // FCHUNK

NWORK = 32
CH = T // NWORK


HD = D // 2


def _router_body(x_ref, wg_ref, bg_ref,
                 logits_ref, pos_ref, meta_ref, wrows_ref, xp_ref):
    x = x_ref[...]
    xb = x.astype(jnp.bfloat16)
    lo = lax.bitcast_convert_type(xb[:, 0:HD], jnp.uint16).astype(jnp.uint32)
    hi = lax.bitcast_convert_type(xb[:, HD:D], jnp.uint16).astype(jnp.uint32)
    xp_ref[...] = lax.bitcast_convert_type(lo | (hi << 16), jnp.int32)
    logits = lax.dot_general(
        x, wg_ref[...], (((1,), (0,)), ((), ())),
        preferred_element_type=jnp.float32) + bg_ref[...]
    logits_ref[...] = logits

    iota = lax.broadcasted_iota(jnp.int32, (T, E), 1)
    m0 = jnp.max(logits, axis=1, keepdims=True)
    e0 = jnp.min(jnp.where(logits == m0, iota, E), axis=1, keepdims=True)
    masked = jnp.where(iota == e0, -jnp.inf, logits)
    m1 = jnp.max(masked, axis=1, keepdims=True)
    e1 = jnp.min(jnp.where(masked == m1, iota, E), axis=1, keepdims=True)
    t = jnp.exp(m1 - m0)
    w0 = 1.0 / (1.0 + t)
    w1 = t / (1.0 + t)

    wrows_ref[0:T, :] = jnp.broadcast_to(w0, (T, 16))
    wrows_ref[T:NP, :] = jnp.broadcast_to(w1, (T, 16))

    r_i = lax.broadcasted_iota(jnp.int32, (CS, CS), 0)
    s_i = lax.broadcasted_iota(jnp.int32, (CS, CS), 1)
    tril = (s_i < r_i).astype(jnp.bfloat16)
    iota_c = lax.broadcasted_iota(jnp.int32, (CS, E), 1)

    counts = jnp.zeros((1, E), jnp.float32)
    rank_cols = [[], []]
    for c in range(NCH):
        kk = c // (T // CS)
        tt = (c % (T // CS)) * CS
        ec = (e0 if kk == 0 else e1)[tt:tt + CS]
        onehot = iota_c == ec
        oh_b = onehot.astype(jnp.bfloat16)
        prefix = lax.dot_general(
            tril, oh_b, (((1,), (0,)), ((), ())),
            preferred_element_type=jnp.float32) + counts
        rank_cols[kk].append(
            jnp.sum(jnp.where(onehot, prefix, 0.0), axis=1, keepdims=True))
        counts = counts + jnp.sum(onehot.astype(jnp.float32), axis=0,
                                  keepdims=True)
    rank0 = jnp.concatenate(rank_cols[0], axis=0)
    rank1 = jnp.concatenate(rank_cols[1], axis=0)

    counts_i = counts.astype(jnp.int32)
    nb = (counts_i + (M - 1)) // M
    e_ri = lax.broadcasted_iota(jnp.int32, (E, E), 0)
    e_si = lax.broadcasted_iota(jnp.int32, (E, E), 1)
    tril8 = (e_ri < e_si).astype(jnp.float32)
    startblk = lax.dot_general(
        nb.astype(jnp.float32), tril8, (((1,), (0,)), ((), ())),
        preferred_element_type=jnp.float32)
    offset = startblk * M
    totalblk = jnp.sum(nb)

    off0 = jnp.sum(jnp.where(iota == e0, offset, 0.0), axis=1, keepdims=True)
    off1 = jnp.sum(jnp.where(iota == e1, offset, 0.0), axis=1, keepdims=True)
    pos0 = off0 + rank0
    pos1 = off1 + rank1
    pos_ref[...] = jnp.concatenate([pos0, pos1], axis=1).astype(jnp.int32)

    b_i = lax.broadcasted_iota(jnp.int32, (NB, E), 0)
    be = jnp.sum((b_i >= startblk.astype(jnp.int32)).astype(jnp.int32),
                 axis=1, keepdims=True) - 1
    be = jnp.clip(be, 0, E - 1)
    b_col = lax.broadcasted_iota(jnp.int32, (NB, 1), 0)
    valid = (b_col < totalblk).astype(jnp.int32)
    bclamp = jnp.minimum(b_col, totalblk.astype(jnp.int32) - 1)
    meta_ref[...] = jnp.concatenate([be, valid, bclamp], axis=1)


def _dispatch_body(x_hbm, posf_hbm, xs_hbm, xbuf, idx, s0, s1):
    wid = lax.axis_index("s") * 2 + lax.axis_index("c")
    base = wid * CH
    pltpu.sync_copy(posf_hbm.at[0, pl.ds(base, CH)], idx.at[0])
    pltpu.sync_copy(posf_hbm.at[1, pl.ds(base, CH)], idx.at[1])
    pltpu.sync_copy(x_hbm.at[pl.ds(base, CH)], xbuf)
    c0 = pltpu.async_copy(xbuf, xs_hbm.at[idx.at[0]], s0)
    c1 = pltpu.async_copy(xbuf, xs_hbm.at[idx.at[1]], s1)
    c0.wait()
    c1.wait()


def _group_mm_body(meta_ref, xs_ref, w1_ref, b1_ref, w2_ref, b2_ref,
                   ys_ref):
    b = pl.program_id(0)
    valid = meta_ref[b, 1]

    @pl.when(valid == 1)
    def _():
        xp = lax.bitcast_convert_type(xs_ref[...], jnp.uint32)
        lo = lax.bitcast_convert_type(
            (xp & 0xFFFF).astype(jnp.uint16), jnp.bfloat16)
        hi = lax.bitcast_convert_type(
            (xp >> 16).astype(jnp.uint16), jnp.bfloat16)
        xs = jnp.concatenate([lo, hi], axis=1)
        y = b2_ref[0] + jnp.zeros((M, D), jnp.float32)
        for f in range(NF):
            w1 = w1_ref[0, :, f * FCHUNK:(f + 1) * FCHUNK].astype(jnp.bfloat16)
            h = lax.dot_general(
                xs, w1, (((1,), (0,)), ((), ())),
                preferred_element_type=jnp.float32)
            h = jnp.maximum(h + b1_ref[0, :, f * FCHUNK:(f + 1) * FCHUNK],
                            0.0).astype(jnp.bfloat16)
            w2 = w2_ref[0, f * FCHUNK:(f + 1) * FCHUNK, :].astype(jnp.bfloat16)
            y = y + lax.dot_general(
                h, w2, (((1,), (0,)), ((), ())),
                preferred_element_type=jnp.float32)
        ys_ref[...] = y


def _combine_body(posf_hbm, ys_hbm, wrows_hbm, out_hbm,
                  idx0, idx1, ya, yb, wbuf0, wbuf1, s0, s1):
    wid = lax.axis_index("s") * 2 + lax.axis_index("c")
    base = wid * CH
    pltpu.sync_copy(posf_hbm.at[0, pl.ds(base, CH)], idx0)
    pltpu.sync_copy(posf_hbm.at[1, pl.ds(base, CH)], idx1)
    c0 = pltpu.async_copy(ys_hbm.at[idx0], ya, s0)
    c1 = pltpu.async_copy(ys_hbm.at[idx1], yb, s1)
    pltpu.sync_copy(wrows_hbm.at[pl.ds(base, CH)], wbuf0)
    pltpu.sync_copy(wrows_hbm.at[pl.ds(T + base, CH)], wbuf1)
    c0.wait()
    c1.wait()

    @pl.loop(0, CH)
    def _(r):
        wa = wbuf0[r, pl.ds(0, 16)]
        wb = wbuf1[r, pl.ds(0, 16)]

        @plsc.parallel_loop(0, D, step=16, unroll=4)
        def _(c):
            ya[r, pl.ds(c, 16)] = (wa * ya[r, pl.ds(c, 16)]
                                   + wb * yb[r, pl.ds(c, 16)])

    pltpu.sync_copy(ya, out_hbm.at[pl.ds(base, CH)])


@jax.jit
def kernel(x, Wg, bg, W1, b1, W2, b2):
    x_flat = x.reshape(T, D)
    logits, pos, meta, wrows, xpack = pl.pallas_call(
        _router_body,
        out_shape=(
            jax.ShapeDtypeStruct((T, E), jnp.float32),
            jax.ShapeDtypeStruct((T, K), jnp.int32),
            jax.ShapeDtypeStruct((NB, 3), jnp.int32),
            jax.ShapeDtypeStruct((NP, 16), jnp.float32),
            jax.ShapeDtypeStruct((T, HD), jnp.int32),
        ),
    )(x_flat, Wg, bg.reshape(1, E))

    posf = pos.T.reshape(K, T)

    mesh = plsc.VectorSubcoreMesh(core_axis_name="c", subcore_axis_name="s")
    xs = pl.kernel(
        _dispatch_body,
        mesh=mesh,
        out_type=jax.ShapeDtypeStruct((PN, HD), jnp.int32),
        scratch_types=[
            pltpu.VMEM((CH, HD), jnp.int32),
            pltpu.VMEM((K, CH), jnp.int32),
            pltpu.SemaphoreType.DMA,
            pltpu.SemaphoreType.DMA,
        ],
    )(xpack, posf)

    ys = pl.pallas_call(
        _group_mm_body,
        grid_spec=pltpu.PrefetchScalarGridSpec(
            num_scalar_prefetch=1,
            grid=(NB,),
            in_specs=[
                pl.BlockSpec((M, HD), lambda b, m: (m[b, 2], 0)),
                pl.BlockSpec((1, D, F), lambda b, m: (m[b, 0], 0, 0)),
                pl.BlockSpec((1, 1, F), lambda b, m: (m[b, 0], 0, 0)),
                pl.BlockSpec((1, F, D), lambda b, m: (m[b, 0], 0, 0)),
                pl.BlockSpec((1, 1, D), lambda b, m: (m[b, 0], 0, 0)),
            ],
            out_specs=pl.BlockSpec((M, D), lambda b, m: (m[b, 2], 0)),
        ),
        out_shape=jax.ShapeDtypeStruct((PN, D), jnp.float32),
        compiler_params=pltpu.CompilerParams(
            dimension_semantics=("parallel",)),
    )(meta, xs, W1, b1.reshape(E, 1, F), W2, b2.reshape(E, 1, D))

    out = pl.kernel(
        _combine_body,
        mesh=mesh,
        out_type=jax.ShapeDtypeStruct((T, D), jnp.float32),
        scratch_types=[
            pltpu.VMEM((CH,), jnp.int32),
            pltpu.VMEM((CH,), jnp.int32),
            pltpu.VMEM((CH, D), jnp.float32),
            pltpu.VMEM((CH, D), jnp.float32),
            pltpu.VMEM((CH, 16), jnp.float32),
            pltpu.VMEM((CH, 16), jnp.float32),
            pltpu.SemaphoreType.DMA,
            pltpu.SemaphoreType.DMA,
        ],
    )(posf, ys, wrows)

    return out.reshape(1, T, D), logits


# speedup vs baseline: 1.3132x; 1.0054x over previous
import functools

import jax
import jax.numpy as jnp
from jax import lax
from jax.experimental import pallas as pl
from jax.experimental.pallas import tpu as pltpu
from jax.experimental.pallas import tpu_sc as plsc

E = 8
K = 2
D = 768
F = 3072
T = 2048
NP = T * K

M = 512
NB = NP // M + E
PN = NB * M

CS = 512
NCH = NP // CS

FCHUNK = 1024
NF = F // FCHUNK

NWORK = 32
CH = T // NWORK


HD = D // 2


def _router_body(x_ref, wg_ref, bg_ref,
                 logits_ref, pos_ref, meta_ref, wrows_ref, xp_ref):
    x = x_ref[...]
    xb = x.astype(jnp.bfloat16)
    lo = lax.bitcast_convert_type(xb[:, 0:HD], jnp.uint16).astype(jnp.uint32)
    hi = lax.bitcast_convert_type(xb[:, HD:D], jnp.uint16).astype(jnp.uint32)
    xp_ref[...] = lax.bitcast_convert_type(lo | (hi << 16), jnp.int32)
    logits = lax.dot_general(
        x, wg_ref[...], (((1,), (0,)), ((), ())),
        preferred_element_type=jnp.float32) + bg_ref[...]
    logits_ref[...] = logits

    iota = lax.broadcasted_iota(jnp.int32, (T, E), 1)
    m0 = jnp.max(logits, axis=1, keepdims=True)
    e0 = jnp.min(jnp.where(logits == m0, iota, E), axis=1, keepdims=True)
    masked = jnp.where(iota == e0, -jnp.inf, logits)
    m1 = jnp.max(masked, axis=1, keepdims=True)
    e1 = jnp.min(jnp.where(masked == m1, iota, E), axis=1, keepdims=True)
    t = jnp.exp(m1 - m0)
    w0 = 1.0 / (1.0 + t)
    w1 = t / (1.0 + t)

    wrows_ref[0:T, :] = jnp.broadcast_to(w0, (T, 16))
    wrows_ref[T:NP, :] = jnp.broadcast_to(w1, (T, 16))

    r_i = lax.broadcasted_iota(jnp.int32, (CS, CS), 0)
    s_i = lax.broadcasted_iota(jnp.int32, (CS, CS), 1)
    tril = (s_i < r_i).astype(jnp.bfloat16)
    iota_c = lax.broadcasted_iota(jnp.int32, (CS, E), 1)

    counts = jnp.zeros((1, E), jnp.float32)
    rank_cols = [[], []]
    for c in range(NCH):
        kk = c // (T // CS)
        tt = (c % (T // CS)) * CS
        ec = (e0 if kk == 0 else e1)[tt:tt + CS]
        onehot = iota_c == ec
        oh_b = onehot.astype(jnp.bfloat16)
        prefix = lax.dot_general(
            tril, oh_b, (((1,), (0,)), ((), ())),
            preferred_element_type=jnp.float32) + counts
        rank_cols[kk].append(
            jnp.sum(jnp.where(onehot, prefix, 0.0), axis=1, keepdims=True))
        counts = counts + jnp.sum(onehot.astype(jnp.float32), axis=0,
                                  keepdims=True)
    rank0 = jnp.concatenate(rank_cols[0], axis=0)
    rank1 = jnp.concatenate(rank_cols[1], axis=0)

    counts_i = counts.astype(jnp.int32)
    nb = (counts_i + (M - 1)) // M
    e_ri = lax.broadcasted_iota(jnp.int32, (E, E), 0)
    e_si = lax.broadcasted_iota(jnp.int32, (E, E), 1)
    tril8 = (e_ri < e_si).astype(jnp.float32)
    startblk = lax.dot_general(
        nb.astype(jnp.float32), tril8, (((1,), (0,)), ((), ())),
        preferred_element_type=jnp.float32)
    offset = startblk * M
    totalblk = jnp.sum(nb)

    off0 = jnp.sum(jnp.where(iota == e0, offset, 0.0), axis=1, keepdims=True)
    off1 = jnp.sum(jnp.where(iota == e1, offset, 0.0), axis=1, keepdims=True)
    pos0 = off0 + rank0
    pos1 = off1 + rank1
    pos_ref[...] = jnp.concatenate([pos0, pos1], axis=1).astype(jnp.int32)

    b_i = lax.broadcasted_iota(jnp.int32, (NB, E), 0)
    be = jnp.sum((b_i >= startblk.astype(jnp.int32)).astype(jnp.int32),
                 axis=1, keepdims=True) - 1
    be = jnp.clip(be, 0, E - 1)
    b_col = lax.broadcasted_iota(jnp.int32, (NB, 1), 0)
    valid = (b_col < totalblk).astype(jnp.int32)
    bclamp = jnp.minimum(b_col, totalblk.astype(jnp.int32) - 1)
    meta_ref[...] = jnp.concatenate([be, valid, bclamp], axis=1)


def _dispatch_body(x_hbm, posf_hbm, xs_hbm, xbuf, idx, s0, s1):
    wid = lax.axis_index("s") * 2 + lax.axis_index("c")
    base = wid * CH
    pltpu.sync_copy(posf_hbm.at[0, pl.ds(base, CH)], idx.at[0])
    pltpu.sync_copy(posf_hbm.at[1, pl.ds(base, CH)], idx.at[1])
    pltpu.sync_copy(x_hbm.at[pl.ds(base, CH)], xbuf)
    c0 = pltpu.async_copy(xbuf, xs_hbm.at[idx.at[0]], s0)
    c1 = pltpu.async_copy(xbuf, xs_hbm.at[idx.at[1]], s1)
    c0.wait()
    c1.wait()


def _group_mm_body(meta_ref, xs_ref, w1_ref, b1_ref, w2_ref, b2_ref,
                   ys_ref):
    b = pl.program_id(0)
    valid = meta_ref[b, 1]

    @pl.when(valid == 1)
    def _():
        xp = lax.bitcast_convert_type(xs_ref[...], jnp.uint32)
        lo = lax.bitcast_convert_type(
            (xp & 0xFFFF).astype(jnp.uint16), jnp.bfloat16)
        hi = lax.bitcast_convert_type(
            (xp >> 16).astype(jnp.uint16), jnp.bfloat16)
        xs = jnp.concatenate([lo, hi], axis=1)
        y = b2_ref[0] + jnp.zeros((M, D), jnp.float32)
        for f in range(NF):
            w1 = w1_ref[0, :, f * FCHUNK:(f + 1) * FCHUNK].astype(jnp.bfloat16)
            h = lax.dot_general(
                xs, w1, (((1,), (0,)), ((), ())),
                preferred_element_type=jnp.float32)
            h = jnp.maximum(h + b1_ref[0, :, f * FCHUNK:(f + 1) * FCHUNK],
                            0.0).astype(jnp.bfloat16)
            w2 = w2_ref[0, f * FCHUNK:(f + 1) * FCHUNK, :].astype(jnp.bfloat16)
            y = y + lax.dot_general(
                h, w2, (((1,), (0,)), ((), ())),
                preferred_element_type=jnp.float32)
        yb16 = y.astype(jnp.bfloat16)
        ylo = lax.bitcast_convert_type(yb16[:, 0:HD],
                                       jnp.uint16).astype(jnp.uint32)
        yhi = lax.bitcast_convert_type(yb16[:, HD:D],
                                       jnp.uint16).astype(jnp.uint32)
        ys_ref[...] = lax.bitcast_convert_type(ylo | (yhi << 16), jnp.int32)


def _gatherback_body(posf_hbm, ys_hbm, yp_hbm, idx0, idx1, ya, yb, s0, s1):
    wid = lax.axis_index("s") * 2 + lax.axis_index("c")
    base = wid * CH
    pltpu.sync_copy(posf_hbm.at[0, pl.ds(base, CH)], idx0)
    pltpu.sync_copy(posf_hbm.at[1, pl.ds(base, CH)], idx1)
    c0 = pltpu.async_copy(ys_hbm.at[idx0], ya, s0)
    c1 = pltpu.async_copy(ys_hbm.at[idx1], yb, s1)
    c0.wait()
    c1.wait()
    pltpu.sync_copy(ya, yp_hbm.at[pl.ds(base, CH)])
    pltpu.sync_copy(yb, yp_hbm.at[pl.ds(T + base, CH)])


def _final_body(yp_ref, wrows_ref, out_ref):
    def unpack(rows):
        xp = lax.bitcast_convert_type(rows, jnp.uint32)
        lo = lax.bitcast_convert_type(
            (xp & 0xFFFF).astype(jnp.uint16), jnp.bfloat16)
        hi = lax.bitcast_convert_type(
            (xp >> 16).astype(jnp.uint16), jnp.bfloat16)
        return jnp.concatenate([lo, hi], axis=1).astype(jnp.float32)

    ya = unpack(yp_ref[0:T, :])
    yb = unpack(yp_ref[T:NP, :])
    w0 = wrows_ref[0:T, 0:1]
    w1 = wrows_ref[T:NP, 0:1]
    out_ref[...] = w0 * ya + w1 * yb


@jax.jit
def kernel(x, Wg, bg, W1, b1, W2, b2):
    x_flat = x.reshape(T, D)
    logits, pos, meta, wrows, xpack = pl.pallas_call(
        _router_body,
        out_shape=(
            jax.ShapeDtypeStruct((T, E), jnp.float32),
            jax.ShapeDtypeStruct((T, K), jnp.int32),
            jax.ShapeDtypeStruct((NB, 3), jnp.int32),
            jax.ShapeDtypeStruct((NP, 16), jnp.float32),
            jax.ShapeDtypeStruct((T, HD), jnp.int32),
        ),
    )(x_flat, Wg, bg.reshape(1, E))

    posf = pos.T.reshape(K, T)

    mesh = plsc.VectorSubcoreMesh(core_axis_name="c", subcore_axis_name="s")
    xs = pl.kernel(
        _dispatch_body,
        mesh=mesh,
        out_type=jax.ShapeDtypeStruct((PN, HD), jnp.int32),
        scratch_types=[
            pltpu.VMEM((CH, HD), jnp.int32),
            pltpu.VMEM((K, CH), jnp.int32),
            pltpu.SemaphoreType.DMA,
            pltpu.SemaphoreType.DMA,
        ],
    )(xpack, posf)

    ys = pl.pallas_call(
        _group_mm_body,
        grid_spec=pltpu.PrefetchScalarGridSpec(
            num_scalar_prefetch=1,
            grid=(NB,),
            in_specs=[
                pl.BlockSpec((M, HD), lambda b, m: (m[b, 2], 0)),
                pl.BlockSpec((1, D, F), lambda b, m: (m[b, 0], 0, 0)),
                pl.BlockSpec((1, 1, F), lambda b, m: (m[b, 0], 0, 0)),
                pl.BlockSpec((1, F, D), lambda b, m: (m[b, 0], 0, 0)),
                pl.BlockSpec((1, 1, D), lambda b, m: (m[b, 0], 0, 0)),
            ],
            out_specs=pl.BlockSpec((M, HD), lambda b, m: (m[b, 2], 0)),
        ),
        out_shape=jax.ShapeDtypeStruct((PN, HD), jnp.int32),
        compiler_params=pltpu.CompilerParams(
            dimension_semantics=("parallel",)),
    )(meta, xs, W1, b1.reshape(E, 1, F), W2, b2.reshape(E, 1, D))

    yp = pl.kernel(
        _gatherback_body,
        mesh=mesh,
        out_type=jax.ShapeDtypeStruct((NP, HD), jnp.int32),
        scratch_types=[
            pltpu.VMEM((CH,), jnp.int32),
            pltpu.VMEM((CH,), jnp.int32),
            pltpu.VMEM((CH, HD), jnp.int32),
            pltpu.VMEM((CH, HD), jnp.int32),
            pltpu.SemaphoreType.DMA,
            pltpu.SemaphoreType.DMA,
        ],
    )(posf, ys)

    out = pl.pallas_call(
        _final_body,
        out_shape=jax.ShapeDtypeStruct((T, D), jnp.float32),
    )(yp, wrows)

    return out.reshape(1, T, D), logits
